# baseline (speedup 1.0000x reference)
import functools

import jax
import jax.numpy as jnp
from jax import lax
from jax.experimental import pallas as pl
from jax.experimental.pallas import tpu as pltpu
from jax.experimental.pallas import tpu_sc as plsc

N = 10000
E = 320000
H = 128
MAXDEG = 64
ZR = 0.8

_B = 128
_NC = 2
_NS = 16
_NW = _NC * _NS
_CW = 80
_EPAD = _NW * _CW * _B
_DT = 5
_DC = 2048
_DPAD = _DT * _DC


def _make_spmm(do_deg):
  mesh = plsc.VectorSubcoreMesh(core_axis_name="c", subcore_axis_name="s")
  out_type = [jax.ShapeDtypeStruct((_NC, N, H), jnp.float32)]
  scratch = [
      pltpu.VMEM((2, _B), jnp.int32),
      pltpu.VMEM((_B * 16,), jnp.float32),
      pltpu.VMEM((_B, H), jnp.float32),
      pltpu.VMEM_SHARED((N, H), jnp.float32),
  ]
  if do_deg:
    out_type.append(jax.ShapeDtypeStruct((_DPAD,), jnp.float32))
    out_type.append(jax.ShapeDtypeStruct((_DPAD,), jnp.float32))
    scratch += [
        pltpu.VMEM((_B,), jnp.float32),
        pltpu.VMEM((_DC,), jnp.float32),
        pltpu.VMEM_SHARED((_DPAD,), jnp.float32),
    ]

  @functools.partial(
      pl.kernel,
      out_type=tuple(out_type),
      mesh=mesh,
      scratch_types=tuple(scratch),
  )
  def spmm(xm_hbm, edata, ewexp, *args):
    if do_deg:
      ewdat = args[0]
      part, deg0, deg1 = args[1:4]
      eb, ewb, rows0, agg_sp, ewsb, zdeg_v, deg_sp = args[4:]
    else:
      ewdat = None
      part = args[0]
      eb, ewb, rows0, agg_sp = args[1:]
      ewsb = None

    cid = lax.axis_index("c")
    sid = lax.axis_index("s")
    wid = cid * _NS + sid

    zero16 = jnp.zeros((16,), jnp.float32)

    def zrow(r, carry):
      for t in range(H // 16):
        rows0[r, pl.ds(t * 16, 16)] = zero16
      return carry

    lax.fori_loop(0, _B, zrow, 0)
    base = pl.multiple_of(sid * 624, 8)

    def _zero_slice(start, nrows):
      for k in range(nrows // _B):
        pltpu.sync_copy(rows0, agg_sp.at[pl.ds(start + k * _B, _B)])
      rem = nrows % _B
      if rem:
        pltpu.sync_copy(rows0.at[pl.ds(0, rem)],
                        agg_sp.at[pl.ds(start + (nrows // _B) * _B, rem)])

    @pl.when(sid < _NS - 1)
    def _():
      _zero_slice(base, 624)

    @pl.when(sid == _NS - 1)
    def _():
      _zero_slice(base, 640)

    if do_deg:
      def zd(i, carry):
        zdeg_v[pl.ds(i * 16, 16)] = zero16
        return carry

      lax.fori_loop(0, _DC // 16, zd, 0)

      @pl.when(sid < _DT)
      def _():
        pltpu.sync_copy(zdeg_v, deg_sp.at[pl.ds(sid * _DC, _DC)])

    plsc.subcore_barrier()

    def _scale():
      @plsc.parallel_loop(0, _B, unroll=4)
      def _(e):
        wv = ewb[pl.ds(e * 16, 16)]
        for t in range(H // 16):
          sl = pl.ds(t * 16, 16)
          rows0[e, sl] = rows0[e, sl] * wv

    def chunk(j, carry):
      pltpu.sync_copy(edata.at[wid, j], eb)
      pltpu.sync_copy(ewexp.at[wid, j], ewb)
      if do_deg:
        pltpu.sync_copy(ewdat.at[wid, j], ewsb)
      pltpu.sync_copy(xm_hbm.at[eb.at[0]], rows0)
      _scale()
      pltpu.sync_copy(rows0, agg_sp.at[eb.at[1]], add=True)
      if do_deg:
        pltpu.sync_copy(ewsb, deg_sp.at[eb.at[1]], add=True)
      return carry

    lax.fori_loop(0, _CW, chunk, 0)
    plsc.subcore_barrier()

    @pl.when(sid < _NS - 1)
    def _():
      pltpu.sync_copy(agg_sp.at[pl.ds(base, 624)],
                      part.at[cid, pl.ds(base, 624)])

    @pl.when(sid == _NS - 1)
    def _():
      pltpu.sync_copy(agg_sp.at[pl.ds(base, 640)],
                      part.at[cid, pl.ds(base, 640)])

    if do_deg:
      @pl.when((sid < _DT) & (cid == 0))
      def _():
        pltpu.sync_copy(deg_sp.at[pl.ds(sid * _DC, _DC)],
                        deg0.at[pl.ds(sid * _DC, _DC)])

      @pl.when((sid < _DT) & (cid == 1))
      def _():
        pltpu.sync_copy(deg_sp.at[pl.ds(sid * _DC, _DC)],
                        deg1.at[pl.ds(sid * _DC, _DC)])

  return spmm


_spmm_deg = _make_spmm(True)
_spmm = _make_spmm(False)


def _gnorm(v, w, b, ms):
  mean = jnp.mean(v, axis=0, keepdims=True)
  out = v - mean * ms
  var = jnp.mean(out * out, axis=0, keepdims=True)
  return w * out * lax.rsqrt(var + 1e-6) + b


def _mmT(a, w):
  return lax.dot_general(a, w, (((1,), (1,)), ((), ())),
                         preferred_element_type=jnp.float32)


def _k0_body(x_ref, z_ref, emb_ref, gw_ref, gb_ref, gms_ref,
             wt1_ref, bt1_ref, wt0_ref, bt0_ref, h_out, xm_out):
  xi = x_ref[...]
  iota = lax.broadcasted_iota(jnp.int32, (N, H), 1)
  oh = (xi == iota).astype(jnp.float32)
  h = jnp.dot(oh, emb_ref[...], preferred_element_type=jnp.float32)
  h = _gnorm(h, gw_ref[...], gb_ref[...], gms_ref[...])
  x1 = jax.nn.relu(_mmT(h, wt1_ref[...]) + bt1_ref[...])
  x0 = jax.nn.relu(_mmT(h, wt0_ref[...]) + bt0_ref[...])
  m = jnp.where(z_ref[...] > 0.5, ZR, 1.0 - ZR)
  h_out[...] = h
  xm_out[...] = x0 + m * (x1 - x0)


def _post_common(p_ref, dpair_ref, z_ref, h_ref, cgn, wc1_ref, bc1_ref,
                 wc0_ref, bc0_ref):
  deg = dpair_ref[:, 0:1] + dpair_ref[:, 1:2]
  deg = jnp.where(deg < 0.5, deg + 1.0, deg)
  agg = (p_ref[0, :, :] + p_ref[1, :, :]) / deg
  agg = _gnorm(agg, *cgn)
  h = h_ref[...]
  wc1 = wc1_ref[...]
  wc0 = wc0_ref[...]
  x1 = _mmT(agg, wc1[:, :H]) + _mmT(h, wc1[:, H:]) + bc1_ref[...]
  x0 = _mmT(agg, wc0[:, :H]) + _mmT(h, wc0[:, H:]) + bc0_ref[...]
  m = jnp.where(z_ref[...] > 0.5, ZR, 1.0 - ZR)
  return x0 + m * (x1 - x0), m


def _k2_body(p_ref, dpair_ref, z_ref, h_ref,
             cgw, cgb, cgms, wc1_ref, bc1_ref, wc0_ref, bc0_ref,
             gw, gb, gms, wt1_ref, bt1_ref, wt0_ref, bt0_ref,
             h_out, xm_out):
  hm, m = _post_common(p_ref, dpair_ref, z_ref, h_ref,
                       (cgw[...], cgb[...], cgms[...]),
                       wc1_ref, bc1_ref, wc0_ref, bc0_ref)
  h1 = jax.nn.relu(_gnorm(hm, gw[...], gb[...], gms[...]))
  y1 = jax.nn.relu(_mmT(h1, wt1_ref[...]) + bt1_ref[...])
  y0 = jax.nn.relu(_mmT(h1, wt0_ref[...]) + bt0_ref[...])
  h_out[...] = h1
  xm_out[...] = y0 + m * (y1 - y0)


def _k4_body(p_ref, dpair_ref, z_ref, h_ref,
             cgw, cgb, cgms, wc1_ref, bc1_ref, wc0_ref, bc0_ref,
             gw, gb, gms, out_ref):
  hm, _ = _post_common(p_ref, dpair_ref, z_ref, h_ref,
                       (cgw[...], cgb[...], cgms[...]),
                       wc1_ref, bc1_ref, wc0_ref, bc0_ref)
  out_ref[...] = _gnorm(hm, gw[...], gb[...], gms[...])


_NH = jax.ShapeDtypeStruct((N, H), jnp.float32)

_k0 = pl.pallas_call(_k0_body, out_shape=(_NH, _NH))
_k2 = pl.pallas_call(_k2_body, out_shape=(_NH, _NH))
_k4 = pl.pallas_call(_k4_body, out_shape=_NH)


@jax.jit
def kernel(x, edge_index, edge_weight, z, params):
  row = edge_index[0].astype(jnp.int32)
  col = edge_index[1].astype(jnp.int32)
  ew = edge_weight.astype(jnp.float32)
  pad = _EPAD - E
  col3 = jnp.pad(col, (0, pad)).reshape(_NW, _CW, _B)
  row3 = jnp.pad(row, (0, pad)).reshape(_NW, _CW, _B)
  ewp = jnp.pad(ew, (0, pad))
  ewdat = ewp.reshape(_NW, _CW, _B)
  ewexp = jnp.broadcast_to(ewp[:, None], (_EPAD, 16)).reshape(
      _NW, _CW, _B * 16)
  edata = jnp.stack([col3, row3], axis=2)
  x2 = x.astype(jnp.int32).reshape(N, 1)
  z2 = z.astype(jnp.float32).reshape(N, 1)
  emb_pad = jnp.zeros((H, H), jnp.float32).at[:MAXDEG + 1].set(params["emb"])

  def v2(t):
    return tuple(a.reshape(1, H) for a in t)

  egw, egb, egms = v2(params["emb_gn"])
  h, xm0 = _k0(x2, z2, emb_pad, egw, egb, egms,
               params["t1_0"][0], params["t1_0"][1].reshape(1, H),
               params["t0_0"][0], params["t0_0"][1].reshape(1, H))

  p0, d0, d1 = _spmm_deg(xm0, edata, ewexp, ewdat)
  dpair = jnp.stack([d0[:N], d1[:N]], axis=1)

  cg0 = v2(params["cgn_0"])
  g0 = v2(params["gn_0"])
  h1, xm1 = _k2(p0, dpair, z2, h,
                cg0[0], cg0[1], cg0[2],
                params["c1_0"][0], params["c1_0"][1].reshape(1, H),
                params["c0_0"][0], params["c0_0"][1].reshape(1, H),
                g0[0], g0[1], g0[2],
                params["t1_1"][0], params["t1_1"][1].reshape(1, H),
                params["t0_1"][0], params["t0_1"][1].reshape(1, H))

  p1, = _spmm(xm1, edata, ewexp)

  cg1 = v2(params["cgn_1"])
  g1 = v2(params["gn_1"])
  out = _k4(p1, dpair, z2, h1,
            cg1[0], cg1[1], cg1[2],
            params["c1_1"][0], params["c1_1"][1].reshape(1, H),
            params["c0_1"][0], params["c0_1"][1].reshape(1, H),
            g1[0], g1[1], g1[2])
  return out


# speedup vs baseline: 1.3232x; 1.3232x over previous
import functools

import jax
import jax.numpy as jnp
from jax import lax
from jax.experimental import pallas as pl
from jax.experimental.pallas import tpu as pltpu
from jax.experimental.pallas import tpu_sc as plsc

N = 10000
E = 320000
H = 128
MAXDEG = 64
ZR = 0.8

_B = 128
_NC = 2
_NS = 16
_NW = _NC * _NS
_CW = 80
_EPAD = _NW * _CW * _B
_DT = 5
_DC = 2048
_DPAD = _DT * _DC


def _make_spmm(do_deg):
  mesh = plsc.VectorSubcoreMesh(core_axis_name="c", subcore_axis_name="s")
  out_type = [jax.ShapeDtypeStruct((_NC, N, H), jnp.float32)]
  scratch = [
      pltpu.VMEM((2, _B), jnp.int32),
      pltpu.VMEM((2, _B), jnp.int32),
      pltpu.VMEM((_B * 16,), jnp.float32),
      pltpu.VMEM((_B * 16,), jnp.float32),
      pltpu.VMEM((_B, H), jnp.float32),
      pltpu.VMEM((_B, H), jnp.float32),
      pltpu.VMEM_SHARED((N, H), jnp.float32),
      pltpu.SemaphoreType.DMA,
      pltpu.SemaphoreType.DMA,
      pltpu.SemaphoreType.DMA,
      pltpu.SemaphoreType.DMA,
  ]
  if do_deg:
    out_type.append(jax.ShapeDtypeStruct((_DPAD,), jnp.float32))
    out_type.append(jax.ShapeDtypeStruct((_DPAD,), jnp.float32))
    scratch += [
        pltpu.VMEM((_B,), jnp.float32),
        pltpu.VMEM((_B,), jnp.float32),
        pltpu.VMEM((_DC,), jnp.float32),
        pltpu.VMEM_SHARED((_DPAD,), jnp.float32),
    ]

  @functools.partial(
      pl.kernel,
      out_type=tuple(out_type),
      mesh=mesh,
      scratch_types=tuple(scratch),
  )
  def spmm(xm_hbm, edata, ewexp, *args):
    if do_deg:
      ewdat = args[0]
      part, deg0, deg1 = args[1:4]
      (eb_a, eb_b, ewb_a, ewb_b, rows_a, rows_b, agg_sp,
       sem_a, sem_b, ssem_a, ssem_b,
       ewsb_a, ewsb_b, zdeg_v, deg_sp) = args[4:]
    else:
      ewdat = None
      part = args[0]
      (eb_a, eb_b, ewb_a, ewb_b, rows_a, rows_b, agg_sp,
       sem_a, sem_b, ssem_a, ssem_b) = args[1:]
      ewsb_a = ewsb_b = None
    rows0 = rows_a

    cid = lax.axis_index("c")
    sid = lax.axis_index("s")
    wid = cid * _NS + sid

    zero16 = jnp.zeros((16,), jnp.float32)

    def zrow(r, carry):
      for t in range(H // 16):
        rows0[r, pl.ds(t * 16, 16)] = zero16
      return carry

    lax.fori_loop(0, _B, zrow, 0)
    base = pl.multiple_of(sid * 624, 8)

    def _zero_slice(start, nrows):
      for k in range(nrows // _B):
        pltpu.sync_copy(rows0, agg_sp.at[pl.ds(start + k * _B, _B)])
      rem = nrows % _B
      if rem:
        pltpu.sync_copy(rows0.at[pl.ds(0, rem)],
                        agg_sp.at[pl.ds(start + (nrows // _B) * _B, rem)])

    @pl.when(sid < _NS - 1)
    def _():
      _zero_slice(base, 624)

    @pl.when(sid == _NS - 1)
    def _():
      _zero_slice(base, 640)

    if do_deg:
      def zd(i, carry):
        zdeg_v[pl.ds(i * 16, 16)] = zero16
        return carry

      lax.fori_loop(0, _DC // 16, zd, 0)

      @pl.when(sid < _DT)
      def _():
        pltpu.sync_copy(zdeg_v, deg_sp.at[pl.ds(sid * _DC, _DC)])

    def _scale(ewb, buf):
      @plsc.parallel_loop(0, _B, unroll=4)
      def _(e):
        wv = ewb[pl.ds(e * 16, 16)]
        for t in range(H // 16):
          sl = pl.ds(t * 16, 16)
          buf[e, sl] = buf[e, sl] * wv

    def _ifetch(j, eb, ewb, ewsb):
      pltpu.sync_copy(edata.at[wid, j], eb)
      pltpu.sync_copy(ewexp.at[wid, j], ewb)
      if do_deg:
        pltpu.sync_copy(ewdat.at[wid, j], ewsb)

    def _gstart(eb, buf, sem):
      pltpu.async_copy(xm_hbm.at[eb.at[0]], buf, sem)

    def _gwait(eb, buf, sem):
      pltpu.make_async_copy(xm_hbm.at[eb.at[0]], buf, sem).wait()

    def _sstart(eb, buf, ssem):
      pltpu.async_copy(buf, agg_sp.at[eb.at[1]], ssem, add=True)

    def _swait(eb, buf, ssem):
      pltpu.make_async_copy(buf, agg_sp.at[eb.at[1]], ssem).wait()

    def _deg_add(eb, ewsb):
      if do_deg:
        pltpu.sync_copy(ewsb, deg_sp.at[eb.at[1]], add=True)

    _A = (eb_a, ewb_a, ewsb_a, rows_a, sem_a, ssem_a)
    _Bu = (eb_b, ewb_b, ewsb_b, rows_b, sem_b, ssem_b)
    _NP = _CW // 2

    _ifetch(0, eb_a, ewb_a, ewsb_a)
    _gstart(eb_a, rows_a, sem_a)

    plsc.subcore_barrier()

    def chunkpair(s, carry):
      j0 = 2 * s
      _ifetch(j0 + 1, eb_b, ewb_b, ewsb_b)
      _gwait(eb_a, rows_a, sem_a)

      @pl.when(s > 0)
      def _():
        _swait(eb_b, rows_b, ssem_b)

      _gstart(eb_b, rows_b, sem_b)
      _scale(ewb_a, rows_a)
      _sstart(eb_a, rows_a, ssem_a)
      _deg_add(eb_a, ewsb_a)

      @pl.when(s < _NP - 1)
      def _():
        _ifetch(j0 + 2, eb_a, ewb_a, ewsb_a)

      _gwait(eb_b, rows_b, sem_b)
      _swait(eb_a, rows_a, ssem_a)

      @pl.when(s < _NP - 1)
      def _():
        _gstart(eb_a, rows_a, sem_a)

      _scale(ewb_b, rows_b)
      _sstart(eb_b, rows_b, ssem_b)
      _deg_add(eb_b, ewsb_b)
      return carry

    lax.fori_loop(0, _NP, chunkpair, 0)
    _swait(eb_b, rows_b, ssem_b)
    plsc.subcore_barrier()

    @pl.when(sid < _NS - 1)
    def _():
      pltpu.sync_copy(agg_sp.at[pl.ds(base, 624)],
                      part.at[cid, pl.ds(base, 624)])

    @pl.when(sid == _NS - 1)
    def _():
      pltpu.sync_copy(agg_sp.at[pl.ds(base, 640)],
                      part.at[cid, pl.ds(base, 640)])

    if do_deg:
      @pl.when((sid < _DT) & (cid == 0))
      def _():
        pltpu.sync_copy(deg_sp.at[pl.ds(sid * _DC, _DC)],
                        deg0.at[pl.ds(sid * _DC, _DC)])

      @pl.when((sid < _DT) & (cid == 1))
      def _():
        pltpu.sync_copy(deg_sp.at[pl.ds(sid * _DC, _DC)],
                        deg1.at[pl.ds(sid * _DC, _DC)])

  return spmm


_spmm_deg = _make_spmm(True)
_spmm = _make_spmm(False)


def _gnorm(v, w, b, ms):
  mean = jnp.mean(v, axis=0, keepdims=True)
  out = v - mean * ms
  var = jnp.mean(out * out, axis=0, keepdims=True)
  return w * out * lax.rsqrt(var + 1e-6) + b


def _mmT(a, w):
  return lax.dot_general(a, w, (((1,), (1,)), ((), ())),
                         preferred_element_type=jnp.float32)


def _k0_body(x_ref, z_ref, emb_ref, gw_ref, gb_ref, gms_ref,
             wt1_ref, bt1_ref, wt0_ref, bt0_ref, h_out, xm_out):
  xi = x_ref[...]
  iota = lax.broadcasted_iota(jnp.int32, (N, H), 1)
  oh = (xi == iota).astype(jnp.float32)
  h = jnp.dot(oh, emb_ref[...], preferred_element_type=jnp.float32)
  h = _gnorm(h, gw_ref[...], gb_ref[...], gms_ref[...])
  x1 = jax.nn.relu(_mmT(h, wt1_ref[...]) + bt1_ref[...])
  x0 = jax.nn.relu(_mmT(h, wt0_ref[...]) + bt0_ref[...])
  m = jnp.where(z_ref[...] > 0.5, ZR, 1.0 - ZR)
  h_out[...] = h
  xm_out[...] = x0 + m * (x1 - x0)


def _post_common(p_ref, dpair_ref, z_ref, h_ref, cgn, wc1_ref, bc1_ref,
                 wc0_ref, bc0_ref):
  deg = dpair_ref[:, 0:1] + dpair_ref[:, 1:2]
  deg = jnp.where(deg < 0.5, deg + 1.0, deg)
  agg = (p_ref[0, :, :] + p_ref[1, :, :]) / deg
  agg = _gnorm(agg, *cgn)
  h = h_ref[...]
  wc1 = wc1_ref[...]
  wc0 = wc0_ref[...]
  x1 = _mmT(agg, wc1[:, :H]) + _mmT(h, wc1[:, H:]) + bc1_ref[...]
  x0 = _mmT(agg, wc0[:, :H]) + _mmT(h, wc0[:, H:]) + bc0_ref[...]
  m = jnp.where(z_ref[...] > 0.5, ZR, 1.0 - ZR)
  return x0 + m * (x1 - x0), m


def _k2_body(p_ref, dpair_ref, z_ref, h_ref,
             cgw, cgb, cgms, wc1_ref, bc1_ref, wc0_ref, bc0_ref,
             gw, gb, gms, wt1_ref, bt1_ref, wt0_ref, bt0_ref,
             h_out, xm_out):
  hm, m = _post_common(p_ref, dpair_ref, z_ref, h_ref,
                       (cgw[...], cgb[...], cgms[...]),
                       wc1_ref, bc1_ref, wc0_ref, bc0_ref)
  h1 = jax.nn.relu(_gnorm(hm, gw[...], gb[...], gms[...]))
  y1 = jax.nn.relu(_mmT(h1, wt1_ref[...]) + bt1_ref[...])
  y0 = jax.nn.relu(_mmT(h1, wt0_ref[...]) + bt0_ref[...])
  h_out[...] = h1
  xm_out[...] = y0 + m * (y1 - y0)


def _k4_body(p_ref, dpair_ref, z_ref, h_ref,
             cgw, cgb, cgms, wc1_ref, bc1_ref, wc0_ref, bc0_ref,
             gw, gb, gms, out_ref):
  hm, _ = _post_common(p_ref, dpair_ref, z_ref, h_ref,
                       (cgw[...], cgb[...], cgms[...]),
                       wc1_ref, bc1_ref, wc0_ref, bc0_ref)
  out_ref[...] = _gnorm(hm, gw[...], gb[...], gms[...])


_NH = jax.ShapeDtypeStruct((N, H), jnp.float32)

_k0 = pl.pallas_call(_k0_body, out_shape=(_NH, _NH))
_k2 = pl.pallas_call(_k2_body, out_shape=(_NH, _NH))
_k4 = pl.pallas_call(_k4_body, out_shape=_NH)


@jax.jit
def kernel(x, edge_index, edge_weight, z, params):
  row = edge_index[0].astype(jnp.int32)
  col = edge_index[1].astype(jnp.int32)
  ew = edge_weight.astype(jnp.float32)
  pad = _EPAD - E
  col3 = jnp.pad(col, (0, pad)).reshape(_NW, _CW, _B)
  row3 = jnp.pad(row, (0, pad)).reshape(_NW, _CW, _B)
  ewp = jnp.pad(ew, (0, pad))
  ewdat = ewp.reshape(_NW, _CW, _B)
  ewexp = jnp.broadcast_to(ewp[:, None], (_EPAD, 16)).reshape(
      _NW, _CW, _B * 16)
  edata = jnp.stack([col3, row3], axis=2)
  x2 = x.astype(jnp.int32).reshape(N, 1)
  z2 = z.astype(jnp.float32).reshape(N, 1)
  emb_pad = jnp.zeros((H, H), jnp.float32).at[:MAXDEG + 1].set(params["emb"])

  def v2(t):
    return tuple(a.reshape(1, H) for a in t)

  egw, egb, egms = v2(params["emb_gn"])
  h, xm0 = _k0(x2, z2, emb_pad, egw, egb, egms,
               params["t1_0"][0], params["t1_0"][1].reshape(1, H),
               params["t0_0"][0], params["t0_0"][1].reshape(1, H))

  p0, d0, d1 = _spmm_deg(xm0, edata, ewexp, ewdat)
  dpair = jnp.stack([d0[:N], d1[:N]], axis=1)

  cg0 = v2(params["cgn_0"])
  g0 = v2(params["gn_0"])
  h1, xm1 = _k2(p0, dpair, z2, h,
                cg0[0], cg0[1], cg0[2],
                params["c1_0"][0], params["c1_0"][1].reshape(1, H),
                params["c0_0"][0], params["c0_0"][1].reshape(1, H),
                g0[0], g0[1], g0[2],
                params["t1_1"][0], params["t1_1"][1].reshape(1, H),
                params["t0_1"][0], params["t0_1"][1].reshape(1, H))

  p1, = _spmm(xm1, edata, ewexp)

  cg1 = v2(params["cgn_1"])
  g1 = v2(params["gn_1"])
  out = _k4(p1, dpair, z2, h1,
            cg1[0], cg1[1], cg1[2],
            params["c1_1"][0], params["c1_1"][1].reshape(1, H),
            params["c0_1"][0], params["c0_1"][1].reshape(1, H),
            g1[0], g1[1], g1[2])
  return out


# speedup vs baseline: 1.3806x; 1.0434x over previous
import functools

import jax
import jax.numpy as jnp
from jax import lax
from jax.experimental import pallas as pl
from jax.experimental.pallas import tpu as pltpu
from jax.experimental.pallas import tpu_sc as plsc

N = 10000
E = 320000
H = 128
MAXDEG = 64
ZR = 0.8

_B = 128
_NC = 2
_NS = 16
_NW = _NC * _NS
_CW = 80
_BLK = 8
_NB = _CW // _BLK
_EPAD = _NW * _CW * _B
_DT = 5
_DC = 2048
_DPAD = _DT * _DC


def _make_spmm(do_deg):
  mesh = plsc.VectorSubcoreMesh(core_axis_name="c", subcore_axis_name="s")
  out_type = [jax.ShapeDtypeStruct((_NC, N, H), jnp.float32)]
  scratch = [
      pltpu.VMEM((_BLK, 2, _B), jnp.int32),
      pltpu.VMEM((_BLK, 2, _B), jnp.int32),
      pltpu.VMEM((_BLK, _B), jnp.float32),
      pltpu.VMEM((_BLK, _B), jnp.float32),
      pltpu.VMEM((_B, H), jnp.float32),
      pltpu.VMEM((_B, H), jnp.float32),
      pltpu.VMEM_SHARED((N, H), jnp.float32),
      pltpu.SemaphoreType.DMA,
      pltpu.SemaphoreType.DMA,
      pltpu.SemaphoreType.DMA,
      pltpu.SemaphoreType.DMA,
      pltpu.SemaphoreType.DMA,
      pltpu.SemaphoreType.DMA,
  ]
  if do_deg:
    out_type.append(jax.ShapeDtypeStruct((_DPAD,), jnp.float32))
    out_type.append(jax.ShapeDtypeStruct((_DPAD,), jnp.float32))
    scratch += [
        pltpu.VMEM((_DC,), jnp.float32),
        pltpu.VMEM_SHARED((_DPAD,), jnp.float32),
    ]

  @functools.partial(
      pl.kernel,
      out_type=tuple(out_type),
      mesh=mesh,
      scratch_types=tuple(scratch),
  )
  def spmm(xm_hbm, edata, ewdat, *refs):
    if do_deg:
      (part, deg0, deg1, eb_x, eb_y, ewb_x, ewb_y, rows_a, rows_b, agg_sp,
       sem_a, sem_b, ssem_a, ssem_b, isem_x, isem_y, zdeg_v, deg_sp) = refs
    else:
      (part, eb_x, eb_y, ewb_x, ewb_y, rows_a, rows_b, agg_sp,
       sem_a, sem_b, ssem_a, ssem_b, isem_x, isem_y) = refs
    cid = lax.axis_index("c")
    sid = lax.axis_index("s")
    wid = cid * _NS + sid

    zero16 = jnp.zeros((16,), jnp.float32)

    def zrow(r, carry):
      for t in range(H // 16):
        rows_a[r, pl.ds(t * 16, 16)] = zero16
      return carry

    lax.fori_loop(0, _B, zrow, 0)
    base = pl.multiple_of(sid * 624, 8)

    def _zero_slice(start, nrows):
      for k in range(nrows // _B):
        pltpu.sync_copy(rows_a, agg_sp.at[pl.ds(start + k * _B, _B)])
      rem = nrows % _B
      if rem:
        pltpu.sync_copy(rows_a.at[pl.ds(0, rem)],
                        agg_sp.at[pl.ds(start + (nrows // _B) * _B, rem)])

    @pl.when(sid < _NS - 1)
    def _():
      _zero_slice(base, 624)

    @pl.when(sid == _NS - 1)
    def _():
      _zero_slice(base, 640)

    if do_deg:
      def zd(i, carry):
        zdeg_v[pl.ds(i * 16, 16)] = zero16
        return carry

      lax.fori_loop(0, _DC // 16, zd, 0)

      @pl.when(sid < _DT)
      def _():
        pltpu.sync_copy(zdeg_v, deg_sp.at[pl.ds(sid * _DC, _DC)])

    def _bfetch(bj, eb, ewb, isem):
      pltpu.async_copy(edata.at[wid, pl.ds(bj * _BLK, _BLK)], eb, isem)
      pltpu.async_copy(ewdat.at[wid, pl.ds(bj * _BLK, _BLK)], ewb, isem)

    def _bwait(bj, eb, ewb, isem):
      pltpu.make_async_copy(edata.at[wid, pl.ds(bj * _BLK, _BLK)],
                            eb, isem).wait()
      pltpu.make_async_copy(ewdat.at[wid, pl.ds(bj * _BLK, _BLK)],
                            ewb, isem).wait()

    def _gstart(idx_ref, buf, sem):
      pltpu.async_copy(xm_hbm.at[idx_ref], buf, sem)

    def _gwait(idx_ref, buf, sem):
      pltpu.make_async_copy(xm_hbm.at[idx_ref], buf, sem).wait()

    def _swait(buf, idx_ref, ssem):
      pltpu.make_async_copy(buf, agg_sp.at[idx_ref], ssem).wait()

    def _scale(ewb, k, buf):
      def scale(g, c2):
        wv16 = ewb[k, pl.ds(g * 16, 16)]
        for i in range(16):
          wv = jnp.full((16,), wv16[i], jnp.float32)
          e = g * 16 + i
          for t in range(H // 16):
            sl = pl.ds(t * 16, 16)
            buf[e, sl] = buf[e, sl] * wv
        return c2

      lax.fori_loop(0, _B // 16, scale, 0)

    _abuf = (rows_a, sem_a, ssem_a)
    _bbuf = (rows_b, sem_b, ssem_b)

    def _do_block(bj, eb, ewb, ebn, ewbn, isem_n, has_next):
      for k in range(_BLK):
        cur, csem, cssem = _abuf if k % 2 == 0 else _bbuf
        nxt, nsem, nssem = _bbuf if k % 2 == 0 else _abuf
        _gwait(eb.at[k, 0], cur, csem)
        if k < _BLK - 1:
          if k == 0:
            @pl.when(bj > 0)
            def _():
              _swait(nxt, eb.at[k, 1], nssem)
          else:
            _swait(nxt, eb.at[k, 1], nssem)
          _gstart(eb.at[k + 1, 0], nxt, nsem)
        else:
          @pl.when(has_next)
          def _():
            _bwait(bj + 1, ebn, ewbn, isem_n)
            _swait(nxt, eb.at[k, 1], nssem)
            _gstart(ebn.at[0, 0], nxt, nsem)
        _scale(ewb, k, cur)
        pltpu.async_copy(cur, agg_sp.at[eb.at[k, 1]], cssem, add=True)
        if do_deg:
          pltpu.sync_copy(ewb.at[k], deg_sp.at[eb.at[k, 1]], add=True)

    _bfetch(0, eb_x, ewb_x, isem_x)
    _bwait(0, eb_x, ewb_x, isem_x)
    _gstart(eb_x.at[0, 0], rows_a, sem_a)
    _bfetch(1, eb_y, ewb_y, isem_y)

    plsc.subcore_barrier()

    def blockpair(s, carry):
      b0 = 2 * s
      _do_block(b0, eb_x, ewb_x, eb_y, ewb_y, isem_y, b0 + 1 <= _NB - 1)

      @pl.when(b0 + 2 <= _NB - 1)
      def _():
        _bfetch(b0 + 2, eb_x, ewb_x, isem_x)

      _do_block(b0 + 1, eb_y, ewb_y, eb_x, ewb_x, isem_x,
                b0 + 2 <= _NB - 1)

      @pl.when(b0 + 3 <= _NB - 1)
      def _():
        _bfetch(b0 + 3, eb_y, ewb_y, isem_y)
      return carry

    lax.fori_loop(0, _NB // 2, blockpair, 0)
    _swait(rows_a, eb_y.at[_BLK - 2, 1], ssem_a)
    _swait(rows_b, eb_y.at[_BLK - 1, 1], ssem_b)
    plsc.subcore_barrier()

    @pl.when(sid < _NS - 1)
    def _():
      pltpu.sync_copy(agg_sp.at[pl.ds(base, 624)],
                      part.at[cid, pl.ds(base, 624)])

    @pl.when(sid == _NS - 1)
    def _():
      pltpu.sync_copy(agg_sp.at[pl.ds(base, 640)],
                      part.at[cid, pl.ds(base, 640)])
    if do_deg:
      @pl.when((sid < _DT) & (cid == 0))
      def _():
        pltpu.sync_copy(deg_sp.at[pl.ds(sid * _DC, _DC)],
                        deg0.at[pl.ds(sid * _DC, _DC)])

      @pl.when((sid < _DT) & (cid == 1))
      def _():
        pltpu.sync_copy(deg_sp.at[pl.ds(sid * _DC, _DC)],
                        deg1.at[pl.ds(sid * _DC, _DC)])

  return spmm


_spmm_deg = _make_spmm(True)
_spmm = _make_spmm(False)


def _gnorm(v, w, b, ms):
  mean = jnp.mean(v, axis=0, keepdims=True)
  out = v - mean * ms
  var = jnp.mean(out * out, axis=0, keepdims=True)
  return w * out * lax.rsqrt(var + 1e-6) + b


def _mmT(a, w):
  return lax.dot_general(a, w, (((1,), (1,)), ((), ())),
                         preferred_element_type=jnp.float32)


def _k0_body(x_ref, z_ref, emb_ref, gw_ref, gb_ref, gms_ref,
             wt1_ref, bt1_ref, wt0_ref, bt0_ref, h_out, xm_out):
  xi = x_ref[...]
  iota = lax.broadcasted_iota(jnp.int32, (N, H), 1)
  oh = (xi == iota).astype(jnp.float32)
  h = jnp.dot(oh, emb_ref[...], preferred_element_type=jnp.float32)
  h = _gnorm(h, gw_ref[...], gb_ref[...], gms_ref[...])
  x1 = jax.nn.relu(_mmT(h, wt1_ref[...]) + bt1_ref[...])
  x0 = jax.nn.relu(_mmT(h, wt0_ref[...]) + bt0_ref[...])
  m = jnp.where(z_ref[...] > 0.5, ZR, 1.0 - ZR)
  h_out[...] = h
  xm_out[...] = x0 + m * (x1 - x0)


def _post_common(p_ref, dpair_ref, z_ref, h_ref, cgn, wc1_ref, bc1_ref,
                 wc0_ref, bc0_ref):
  deg = dpair_ref[:, 0:1] + dpair_ref[:, 1:2]
  deg = jnp.where(deg < 0.5, deg + 1.0, deg)
  agg = (p_ref[0, :, :] + p_ref[1, :, :]) / deg
  agg = _gnorm(agg, *cgn)
  h = h_ref[...]
  wc1 = wc1_ref[...]
  wc0 = wc0_ref[...]
  x1 = _mmT(agg, wc1[:, :H]) + _mmT(h, wc1[:, H:]) + bc1_ref[...]
  x0 = _mmT(agg, wc0[:, :H]) + _mmT(h, wc0[:, H:]) + bc0_ref[...]
  m = jnp.where(z_ref[...] > 0.5, ZR, 1.0 - ZR)
  return x0 + m * (x1 - x0), m


def _k2_body(p_ref, dpair_ref, z_ref, h_ref,
             cgw, cgb, cgms, wc1_ref, bc1_ref, wc0_ref, bc0_ref,
             gw, gb, gms, wt1_ref, bt1_ref, wt0_ref, bt0_ref,
             h_out, xm_out):
  hm, m = _post_common(p_ref, dpair_ref, z_ref, h_ref,
                       (cgw[...], cgb[...], cgms[...]),
                       wc1_ref, bc1_ref, wc0_ref, bc0_ref)
  h1 = jax.nn.relu(_gnorm(hm, gw[...], gb[...], gms[...]))
  y1 = jax.nn.relu(_mmT(h1, wt1_ref[...]) + bt1_ref[...])
  y0 = jax.nn.relu(_mmT(h1, wt0_ref[...]) + bt0_ref[...])
  h_out[...] = h1
  xm_out[...] = y0 + m * (y1 - y0)


def _k4_body(p_ref, dpair_ref, z_ref, h_ref,
             cgw, cgb, cgms, wc1_ref, bc1_ref, wc0_ref, bc0_ref,
             gw, gb, gms, out_ref):
  hm, _ = _post_common(p_ref, dpair_ref, z_ref, h_ref,
                       (cgw[...], cgb[...], cgms[...]),
                       wc1_ref, bc1_ref, wc0_ref, bc0_ref)
  out_ref[...] = _gnorm(hm, gw[...], gb[...], gms[...])


_NH = jax.ShapeDtypeStruct((N, H), jnp.float32)

_k0 = pl.pallas_call(_k0_body, out_shape=(_NH, _NH))
_k2 = pl.pallas_call(_k2_body, out_shape=(_NH, _NH))
_k4 = pl.pallas_call(_k4_body, out_shape=_NH)


@jax.jit
def kernel(x, edge_index, edge_weight, z, params):
  row = edge_index[0].astype(jnp.int32)
  col = edge_index[1].astype(jnp.int32)
  ew = edge_weight.astype(jnp.float32)
  pad = _EPAD - E
  col3 = jnp.pad(col, (0, pad)).reshape(_NW, _CW, _B)
  row3 = jnp.pad(row, (0, pad)).reshape(_NW, _CW, _B)
  ewdat = jnp.pad(ew, (0, pad)).reshape(_NW, _CW, _B)
  edata = jnp.stack([col3, row3], axis=2)
  x2 = x.astype(jnp.int32).reshape(N, 1)
  z2 = z.astype(jnp.float32).reshape(N, 1)
  emb_pad = jnp.zeros((H, H), jnp.float32).at[:MAXDEG + 1].set(params["emb"])

  def v2(t):
    return tuple(a.reshape(1, H) for a in t)

  egw, egb, egms = v2(params["emb_gn"])
  h, xm0 = _k0(x2, z2, emb_pad, egw, egb, egms,
               params["t1_0"][0], params["t1_0"][1].reshape(1, H),
               params["t0_0"][0], params["t0_0"][1].reshape(1, H))

  p0, d0, d1 = _spmm_deg(xm0, edata, ewdat)
  dpair = jnp.stack([d0[:N], d1[:N]], axis=1)

  cg0 = v2(params["cgn_0"])
  g0 = v2(params["gn_0"])
  h1, xm1 = _k2(p0, dpair, z2, h,
                cg0[0], cg0[1], cg0[2],
                params["c1_0"][0], params["c1_0"][1].reshape(1, H),
                params["c0_0"][0], params["c0_0"][1].reshape(1, H),
                g0[0], g0[1], g0[2],
                params["t1_1"][0], params["t1_1"][1].reshape(1, H),
                params["t0_1"][0], params["t0_1"][1].reshape(1, H))

  p1, = _spmm(xm1, edata, ewdat)

  cg1 = v2(params["cgn_1"])
  g1 = v2(params["gn_1"])
  out = _k4(p1, dpair, z2, h1,
            cg1[0], cg1[1], cg1[2],
            params["c1_1"][0], params["c1_1"][1].reshape(1, H),
            params["c0_1"][0], params["c0_1"][1].reshape(1, H),
            g1[0], g1[1], g1[2])
  return out
